# baseline (speedup 1.0000x reference)
import functools

import jax
import jax.numpy as jnp
from jax import lax
from jax.experimental import pallas as pl
from jax.experimental.pallas import tpu as pltpu
from jax.experimental.pallas import tpu_sc as plsc

B = 16384
D = 64
K = 20
J = K + 2
V = 1000000
NC = 2
NS = 16
NW = NC * NS
BPW = B // NW
C = 16
NCHUNK = BPW // C
G = C // 16
TQ = C * J // 16


def _sc_scores_kernel(gidx_hbm, comb2_hbm, out_hbm,
                      gidx_v, half_v, rows22, scores, sem_c):
    wid = lax.axis_index("s") * NC + lax.axis_index("c")
    base = wid * BPW
    iota = lax.iota(jnp.int32, 16)

    def chunk_body(ci, carry):
        off = base + ci * C
        pltpu.sync_copy(gidx_hbm.at[pl.ds(off * J, C * J)], gidx_v)

        for q in range(TQ):
            v = gidx_v[pl.ds(q * 16, 16)]
            half_v[pl.ds(q * 16, 16)] = lax.shift_right_logical(v, 1)

        pltpu.async_copy(comb2_hbm.at[half_v], rows22, sem_c).wait()

        for g in range(G):
            rows = g * 16 + iota
            srow = rows * J
            zero = jnp.zeros((16,), jnp.float32)
            for sel in ([0] + list(range(1, 11)), [0] + list(range(11, J))):
                rowv, parc = [], []
                for j in sel:
                    gvec = plsc.load_gather(gidx_v, [srow + j])
                    rowv.append(srow + j)
                    parc.append((gvec & 1) * 64)

                def dbody(d, accs, rowv=rowv, parc=parc):
                    dvec = jnp.full((16,), 0, jnp.int32) + d
                    tcol = plsc.load_gather(rows22, [rowv[0], parc[0] + dvec])
                    new = []
                    for i in range(1, len(rowv)):
                        val = plsc.load_gather(rows22,
                                               [rowv[i], parc[i] + dvec])
                        new.append(accs[i - 1] + tcol * val)
                    return tuple(new)

                accs = lax.fori_loop(0, D, dbody, (zero,) * (len(sel) - 1))
                for i, j in enumerate(sel[1:]):
                    plsc.store_scatter(
                        scores, [rows, jnp.full((16,), j - 1, jnp.int32)],
                        accs[i])

        pltpu.sync_copy(scores, out_hbm.at[pl.ds(off, C)])
        return carry

    lax.fori_loop(0, NCHUNK, chunk_body, 0)


_sc_scores = functools.partial(
    pl.kernel,
    mesh=plsc.VectorSubcoreMesh(core_axis_name="c", subcore_axis_name="s"),
    compiler_params=pltpu.CompilerParams(needs_layout_passes=False),
    out_type=jax.ShapeDtypeStruct((B, J - 1), jnp.float32),
    scratch_types=[
        pltpu.VMEM((C * J,), jnp.int32),
        pltpu.VMEM((C * J,), jnp.int32),
        pltpu.VMEM((C * J, 128), jnp.float32),
        pltpu.VMEM((C, J - 1), jnp.float32),
        pltpu.SemaphoreType.DMA,
    ],
)(_sc_scores_kernel)


def _loss_kernel(scores_ref, out_ref):
    s = scores_ref[...]
    pos = s[:, 0]
    neg = s[:, 1:]
    per_b = -jax.nn.log_sigmoid(pos) - jnp.sum(jax.nn.log_sigmoid(-neg), axis=1)
    out_ref[0, 0] = jnp.sum(per_b) * (1.0 / B)


def kernel(target_idx, context_idx, negative_indices, target_table, context_table):
    comb2 = jnp.concatenate([target_table, context_table], axis=0).reshape(V, 128)
    gidx = jnp.concatenate(
        [target_idx.astype(jnp.int32)[:, None],
         context_idx.astype(jnp.int32)[:, None] + V,
         negative_indices.astype(jnp.int32) + V], axis=1).reshape(-1)
    scores = _sc_scores(gidx, comb2)
    loss = pl.pallas_call(
        _loss_kernel,
        out_shape=jax.ShapeDtypeStruct((1, 1), jnp.float32),
        in_specs=[pl.BlockSpec(memory_space=pltpu.VMEM)],
        out_specs=pl.BlockSpec(memory_space=pltpu.SMEM),
    )(scores)
    return loss[0, 0]


# speedup vs baseline: 1.1957x; 1.1957x over previous
import functools

import jax
import jax.numpy as jnp
from jax import lax
from jax.experimental import pallas as pl
from jax.experimental.pallas import tpu as pltpu
from jax.experimental.pallas import tpu_sc as plsc

B = 16384
D = 64
K = 20
J = K + 2
V = 1000000
NC = 2
NS = 16
NW = NC * NS
BPW = B // NW
C = 16
NCHUNK = BPW // C
TQ = C * J // 16


def _sc_scores_kernel(gidx_hbm, comb2_hbm, out_hbm,
                      gidx_v, half_v, rows22, scores, sem_c):
    wid = lax.axis_index("s") * NC + lax.axis_index("c")
    base = wid * BPW
    iota = lax.iota(jnp.int32, 16)

    def chunk_body(ci, carry):
        off = base + ci * C
        pltpu.sync_copy(gidx_hbm.at[pl.ds(off * J, C * J)], gidx_v)

        for q in range(TQ):
            v = gidx_v[pl.ds(q * 16, 16)]
            half_v[pl.ds(q * 16, 16)] = lax.shift_right_logical(v, 1)

        pltpu.async_copy(comb2_hbm.at[half_v], rows22, sem_c).wait()

        zero = jnp.zeros((16,), jnp.float32)

        def bbody(b, svecs):
            bmask = iota == b
            parA = gidx_v[pl.ds(b * J, 16)] & 1
            parB = gidx_v[pl.ds(b * J + 16, 16)] & 1

            def halves(j):
                row = b * J + j
                par = parA[j] if j < 16 else parB[j - 16]
                m = jnp.full((16,), par, jnp.int32) == 1
                out = []
                for k in range(4):
                    lo = rows22[row, pl.ds(16 * k, 16)]
                    hi = rows22[row, pl.ds(64 + 16 * k, 16)]
                    out.append(jnp.where(m, hi, lo))
                return out

            t = halves(0)
            new = []
            for j in range(1, J):
                n = halves(j)
                acc = t[0] * n[0] + t[1] * n[1] + t[2] * n[2] + t[3] * n[3]
                new.append(jnp.where(bmask, jnp.sum(acc), svecs[j - 1]))
            return tuple(new)

        svecs = lax.fori_loop(0, C, bbody, (zero,) * (J - 1))
        for j in range(J - 1):
            plsc.store_scatter(scores, [iota, jnp.full((16,), j, jnp.int32)],
                               svecs[j])

        pltpu.sync_copy(scores, out_hbm.at[pl.ds(off, C)])
        return carry

    lax.fori_loop(0, NCHUNK, chunk_body, 0)


_sc_scores = functools.partial(
    pl.kernel,
    mesh=plsc.VectorSubcoreMesh(core_axis_name="c", subcore_axis_name="s"),
    compiler_params=pltpu.CompilerParams(needs_layout_passes=False),
    out_type=jax.ShapeDtypeStruct((B, J - 1), jnp.float32),
    scratch_types=[
        pltpu.VMEM((C * J,), jnp.int32),
        pltpu.VMEM((C * J,), jnp.int32),
        pltpu.VMEM((C * J, 128), jnp.float32),
        pltpu.VMEM((C, J - 1), jnp.float32),
        pltpu.SemaphoreType.DMA,
    ],
)(_sc_scores_kernel)


def _loss_kernel(scores_ref, out_ref):
    s = scores_ref[...]
    pos = s[:, 0]
    neg = s[:, 1:]
    per_b = -jax.nn.log_sigmoid(pos) - jnp.sum(jax.nn.log_sigmoid(-neg), axis=1)
    out_ref[0, 0] = jnp.sum(per_b) * (1.0 / B)


def kernel(target_idx, context_idx, negative_indices, target_table, context_table):
    comb2 = jnp.concatenate([target_table, context_table], axis=0).reshape(V, 128)
    gidx = jnp.concatenate(
        [target_idx.astype(jnp.int32)[:, None],
         context_idx.astype(jnp.int32)[:, None] + V,
         negative_indices.astype(jnp.int32) + V], axis=1).reshape(-1)
    scores = _sc_scores(gidx, comb2)
    loss = pl.pallas_call(
        _loss_kernel,
        out_shape=jax.ShapeDtypeStruct((1, 1), jnp.float32),
        in_specs=[pl.BlockSpec(memory_space=pltpu.VMEM)],
        out_specs=pl.BlockSpec(memory_space=pltpu.SMEM),
    )(scores)
    return loss[0, 0]


# speedup vs baseline: 2.4135x; 2.0184x over previous
import functools

import jax
import jax.numpy as jnp
from jax import lax
from jax.experimental import pallas as pl
from jax.experimental.pallas import tpu as pltpu
from jax.experimental.pallas import tpu_sc as plsc

B = 16384
D = 64
K = 20
NC = 2
NS = 16
NW = NC * NS
BPW = B // NW
C = 32
NCHUNK = BPW // C
G = C // 16
NGRP = C * K // 16


def _sc_scores_kernel(tidx_hbm, cidx_hbm, nidx_hbm, ttab_hbm, ctab_hbm,
                      out_hbm, tidx_v, cidx_v, nidx_v, trows, crows, nrows,
                      scores, sem_t, sem_c, sem_n):
    wid = lax.axis_index("s") * NC + lax.axis_index("c")
    base = wid * BPW
    iota = lax.iota(jnp.int32, 16)

    def chunk_body(ci, carry):
        off = base + ci * C
        pltpu.sync_copy(tidx_hbm.at[pl.ds(off, C)], tidx_v)
        pltpu.sync_copy(cidx_hbm.at[pl.ds(off, C)], cidx_v)
        pltpu.sync_copy(nidx_hbm.at[pl.ds(off * K, C * K)], nidx_v)

        for g in range(G):
            tvec = tidx_v[pl.ds(g * 16, 16)]
            cvec = cidx_v[pl.ds(g * 16, 16)]
            for j in range(16):
                pltpu.async_copy(ttab_hbm.at[tvec[j]], trows.at[g * 16 + j],
                                 sem_t)
                pltpu.async_copy(ctab_hbm.at[cvec[j]], crows.at[g * 16 + j],
                                 sem_c)

        def neg_issue(g, carry2):
            nvec = nidx_v[pl.ds(g * 16, 16)]
            for j in range(16):
                pltpu.async_copy(ctab_hbm.at[nvec[j]], nrows.at[g * 16 + j],
                                 sem_n)
            return carry2

        lax.fori_loop(0, NGRP, neg_issue, 0)

        pltpu.make_async_copy(ttab_hbm.at[pl.ds(0, C)], trows, sem_t).wait()
        pltpu.make_async_copy(ctab_hbm.at[pl.ds(0, C)], crows, sem_c).wait()
        pltpu.make_async_copy(ctab_hbm.at[pl.ds(0, C * K)], nrows, sem_n).wait()

        zero = jnp.zeros((16,), jnp.float32)
        for g in range(G):
            def bbody(b, svecs, g=g):
                bmask = iota == (b - g * 16)
                t = [trows[b, pl.ds(16 * k, 16)] for k in range(4)]
                c = [crows[b, pl.ds(16 * k, 16)] for k in range(4)]
                acc = t[0] * c[0] + t[1] * c[1] + t[2] * c[2] + t[3] * c[3]
                new = [jnp.where(bmask, jnp.sum(acc), svecs[0])]
                for k in range(K):
                    r = b * K + k
                    acc = t[0] * nrows[r, pl.ds(0, 16)]
                    acc += t[1] * nrows[r, pl.ds(16, 16)]
                    acc += t[2] * nrows[r, pl.ds(32, 16)]
                    acc += t[3] * nrows[r, pl.ds(48, 16)]
                    new.append(jnp.where(bmask, jnp.sum(acc), svecs[k + 1]))
                return tuple(new)

            svecs = lax.fori_loop(g * 16, g * 16 + 16, bbody, (zero,) * (K + 1))
            rows = g * 16 + iota
            for k in range(K + 1):
                plsc.store_scatter(scores, [rows, jnp.full((16,), k, jnp.int32)],
                                   svecs[k])

        pltpu.sync_copy(scores, out_hbm.at[pl.ds(off, C)])
        return carry

    lax.fori_loop(0, NCHUNK, chunk_body, 0)


_sc_scores = functools.partial(
    pl.kernel,
    mesh=plsc.VectorSubcoreMesh(core_axis_name="c", subcore_axis_name="s"),
    compiler_params=pltpu.CompilerParams(needs_layout_passes=False),
    out_type=jax.ShapeDtypeStruct((B, K + 1), jnp.float32),
    scratch_types=[
        pltpu.VMEM((C,), jnp.int32),
        pltpu.VMEM((C,), jnp.int32),
        pltpu.VMEM((C * K,), jnp.int32),
        pltpu.VMEM((C, D), jnp.float32),
        pltpu.VMEM((C, D), jnp.float32),
        pltpu.VMEM((C * K, D), jnp.float32),
        pltpu.VMEM((C, K + 1), jnp.float32),
        pltpu.SemaphoreType.DMA,
        pltpu.SemaphoreType.DMA,
        pltpu.SemaphoreType.DMA,
    ],
)(_sc_scores_kernel)


def _loss_kernel(scores_ref, out_ref):
    s = scores_ref[...]
    pos = s[:, 0]
    neg = s[:, 1:]
    per_b = -jax.nn.log_sigmoid(pos) - jnp.sum(jax.nn.log_sigmoid(-neg), axis=1)
    out_ref[0, 0] = jnp.sum(per_b) * (1.0 / B)


def kernel(target_idx, context_idx, negative_indices, target_table, context_table):
    scores = _sc_scores(target_idx.astype(jnp.int32),
                        context_idx.astype(jnp.int32),
                        negative_indices.astype(jnp.int32).reshape(-1),
                        target_table, context_table)
    loss = pl.pallas_call(
        _loss_kernel,
        out_shape=jax.ShapeDtypeStruct((1, 1), jnp.float32),
        in_specs=[pl.BlockSpec(memory_space=pltpu.VMEM)],
        out_specs=pl.BlockSpec(memory_space=pltpu.SMEM),
    )(scores)
    return loss[0, 0]
